# baseline (speedup 1.0000x reference)
import functools

import jax
import jax.numpy as jnp
from jax import lax
from jax.experimental import pallas as pl
from jax.experimental.pallas import tpu as pltpu
from jax.experimental.pallas import tpu_sc as plsc

B, S, D = 4096, 200, 256
NTYPE, NIDX = 5, 50
NCOMBO = NTYPE * NIDX
NC, NS = 2, 16
NW = NC * NS
NB = B // NW
LANES = 16
BLK = 50
NBLK = S // BLK
CPAD = 216
TABF = NCOMBO * D
PATF = NIDX * D
PAGE0 = NIDX * D
BLKF = BLK * D


def _style_mask_body(types_ref, inds_ref, sv_ref, w1_ref, b1_ref, w2_ref,
                     b2_ref, temb_ref, iemb_ref,
                     styled_ref, mask_ref, combo_ref, cid_ref):
    h = jnp.dot(sv_ref[...], w1_ref[...], preferred_element_type=jnp.float32)
    h = jnp.maximum(h + b1_ref[...][None, :], 0.0)
    styled = jnp.dot(h, w2_ref[...], preferred_element_type=jnp.float32)
    styled_ref[...] = styled + b2_ref[...][None, :]
    mask_ref[...] = types_ref[...] == 0
    combo_ref[...] = (temb_ref[...][:, None, :]
                      + iemb_ref[...][None, :, :]).reshape(NCOMBO, D)
    cid_ref[...] = types_ref[...] * NIDX + inds_ref[...]


def _tc_pre(types, inds, style_vector, w1, b1, w2, b2, temb, iemb):
    return pl.pallas_call(
        _style_mask_body,
        out_shape=[
            jax.ShapeDtypeStruct((B, D), jnp.float32),
            jax.ShapeDtypeStruct((B, S), jnp.bool_),
            jax.ShapeDtypeStruct((NCOMBO, D), jnp.float32),
            jax.ShapeDtypeStruct((B, S), jnp.int32),
        ],
    )(types, inds, style_vector, w1, b1, w2, b2, temb, iemb)


@functools.partial(
    pl.kernel,
    out_type=jax.ShapeDtypeStruct((B * S * D,), jnp.float32),
    mesh=plsc.VectorSubcoreMesh(
        core_axis_name="c", subcore_axis_name="s",
        num_cores=NC, num_subcores=NS),
    compiler_params=pltpu.CompilerParams(needs_layout_passes=False),
    scratch_types=[
        pltpu.VMEM((TABF + PATF,), jnp.float32),
        pltpu.VMEM((NBLK * BLKF,), jnp.float32),
        pltpu.VMEM((2 * CPAD,), jnp.int32),
        pltpu.VMEM((2 * D,), jnp.float32),
        pltpu.SemaphoreType.DMA,
        pltpu.SemaphoreType.DMA,
        pltpu.SemaphoreType.DMA,
        pltpu.SemaphoreType.DMA,
        pltpu.SemaphoreType.DMA,
        pltpu.SemaphoreType.DMA,
        pltpu.SemaphoreType.DMA,
        pltpu.SemaphoreType.DMA,
        pltpu.SemaphoreType.DMA,
    ],
)
def _sc_embed(cid_hbm, styled_hbm, combo_hbm, pemb_hbm, out_hbm,
              tab, outb, cidb, styb,
              ini0, ini1, ini2, ini3, out0, out1, out2, out3, fsem):
    ini = (ini0, ini1, ini2, ini3)
    osem = (out0, out1, out2, out3)
    wid = lax.axis_index("s") * NC + lax.axis_index("c")

    pltpu.sync_copy(combo_hbm, tab.at[pl.ds(0, TABF)])
    pltpu.sync_copy(cid_hbm.at[pl.ds(wid * NB * S, S)], cidb.at[pl.ds(0, S)])
    pltpu.sync_copy(styled_hbm.at[pl.ds(wid * NB * D, D)],
                    styb.at[pl.ds(0, D)])
    pltpu.async_copy(pemb_hbm.at[pl.ds(0 * BLKF, BLKF)],
                     outb.at[pl.ds(0 * BLKF, BLKF)], ini[0])
    pltpu.async_copy(pemb_hbm.at[pl.ds(1 * BLKF, BLKF)],
                     outb.at[pl.ds(1 * BLKF, BLKF)], ini[1])

    iota = lax.iota(jnp.int32, LANES)
    sflat = tuple(tuple((iota + w * LANES) * D + q * BLKF for w in range(4))
                  for q in range(NBLK))
    tail = BLK - 3 * LANES
    masks = (None, None, None, iota < tail)

    def row_body(r, carry):
        b = wid * NB + r
        slot = r % 2

        @pl.when(r + 1 < NB)
        def _():
            nslot = (r + 1) % 2
            pltpu.async_copy(cid_hbm.at[pl.ds((b + 1) * S, S)],
                             cidb.at[pl.ds(nslot * CPAD, S)], fsem)
            pltpu.async_copy(styled_hbm.at[pl.ds((b + 1) * D, D)],
                             styb.at[pl.ds(nslot * D, D)], fsem)

        @pl.when(r >= 1)
        def _():
            pltpu.make_async_copy(cid_hbm.at[pl.ds(b * S, S)],
                                  cidb.at[pl.ds(slot * CPAD, S)], fsem).wait()
            pltpu.make_async_copy(styled_hbm.at[pl.ds(b * D, D)],
                                  styb.at[pl.ds(slot * D, D)], fsem).wait()

        sty = tuple(styb[pl.ds(slot * D + j * LANES, LANES)]
                    for j in range(D // LANES))

        def patch_body(rr, sty_c):
            base = rr * D
            for j in range(D // LANES):
                v = tab[pl.ds(PAGE0 + base + j * LANES, LANES)] + sty_c[j]
                tab[pl.ds(TABF + base + j * LANES, LANES)] = v
            return sty_c

        lax.fori_loop(0, NIDX, patch_body, sty)

        for q in range(NBLK):
            pltpu.make_async_copy(pemb_hbm.at[pl.ds(q * BLKF, BLKF)],
                                  outb.at[pl.ds(q * BLKF, BLKF)],
                                  ini[q]).wait()

            cvecs = []
            for w in range(4):
                cv = cidb[pl.ds(slot * CPAD + q * BLK + w * LANES, LANES)]
                page = (cv >= NIDX) & (cv < 2 * NIDX)
                cvecs.append(cv * D + jnp.where(page, TABF - PAGE0, 0))
            cvecs = tuple(cvecs)

            def col_body(ci, carry2, _q=q, _cvecs=cvecs):
                for u in range(2):
                    c = ci * 2 + u
                    colv = iota * 0 + c
                    for w in range(4):
                        g = plsc.load_gather(
                            tab, [_cvecs[w] + colv], mask=masks[w])
                        plsc.addupdate_scatter(
                            outb, [sflat[_q][w] + colv], g, mask=masks[w])
                return carry2

            lax.fori_loop(0, D // 2, col_body, 0)

            pltpu.async_copy(
                outb.at[pl.ds(q * BLKF, BLKF)],
                out_hbm.at[pl.ds(b * S * D + q * BLKF, BLKF)], osem[q])

            q2 = (q + 2) % NBLK
            if q >= 2:
                pltpu.make_async_copy(
                    outb.at[pl.ds(q2 * BLKF, BLKF)],
                    out_hbm.at[pl.ds(b * S * D + q2 * BLKF, BLKF)],
                    osem[q2]).wait()

                @pl.when(r + 1 < NB)
                def _(_q2=q2):
                    pltpu.async_copy(pemb_hbm.at[pl.ds(_q2 * BLKF, BLKF)],
                                     outb.at[pl.ds(_q2 * BLKF, BLKF)],
                                     ini[_q2])
            else:
                @pl.when(r >= 1)
                def _(_q2=q2, _b=b):
                    pltpu.make_async_copy(
                        outb.at[pl.ds(_q2 * BLKF, BLKF)],
                        out_hbm.at[pl.ds(_b * S * D + _q2 * BLKF, BLKF)],
                        osem[_q2]).wait()

                pltpu.async_copy(pemb_hbm.at[pl.ds(q2 * BLKF, BLKF)],
                                 outb.at[pl.ds(q2 * BLKF, BLKF)], ini[q2])
        return carry

    lax.fori_loop(0, NB, row_body, 0)

    blast = wid * NB + NB - 1
    for q2 in (2, 3):
        pltpu.make_async_copy(
            outb.at[pl.ds(q2 * BLKF, BLKF)],
            out_hbm.at[pl.ds(blast * S * D + q2 * BLKF, BLKF)],
            osem[q2]).wait()


def kernel(element_types, element_indices, style_vector, type_emb, idx_emb,
           W1, b1, W2, b2, pos_emb):
    types = element_types.astype(jnp.int32)
    inds = element_indices.astype(jnp.int32)
    styled, mask, combo, cid = _tc_pre(
        types, inds, style_vector, W1, b1, W2, b2, type_emb, idx_emb)
    final = _sc_embed(cid.reshape(-1), styled.reshape(-1), combo.reshape(-1),
                      pos_emb.reshape(-1))
    return final.reshape(B, S, D), mask


# speedup vs baseline: 14.6378x; 14.6378x over previous
import functools

import jax
import jax.numpy as jnp
from jax import lax
from jax.experimental import pallas as pl
from jax.experimental.pallas import tpu as pltpu
from jax.experimental.pallas import tpu_sc as plsc

B, S, D = 4096, 200, 256
NTYPE, NIDX = 5, 50
NCOMBO = NTYPE * NIDX
NTAB = NCOMBO * S
NC, NS = 2, 16
NW = NC * NS
NB = B // NW
LANES = 16
G0, G1 = 104, 96
MSTR = 208
PAGE_LO, PAGE_HI = NIDX * S, 2 * NIDX * S


def _style_mask_body(types_ref, inds_ref, sv_ref, w1_ref, b1_ref, w2_ref,
                     b2_ref, styled_ref, mask_ref, gidx_ref):
    h = jnp.dot(sv_ref[...], w1_ref[...], preferred_element_type=jnp.float32)
    h = jnp.maximum(h + b1_ref[...][None, :], 0.0)
    styled = jnp.dot(h, w2_ref[...], preferred_element_type=jnp.float32)
    styled_ref[...] = styled + b2_ref[...][None, :]
    mask_ref[...] = types_ref[...] == 0
    s_iota = lax.broadcasted_iota(jnp.int32, (B, S), 1)
    gidx_ref[...] = (types_ref[...] * NIDX + inds_ref[...]) * S + s_iota


def _tc_pre(types, inds, style_vector, w1, b1, w2, b2):
    return pl.pallas_call(
        _style_mask_body,
        out_shape=[
            jax.ShapeDtypeStruct((B, D), jnp.float32),
            jax.ShapeDtypeStruct((B, S), jnp.bool_),
            jax.ShapeDtypeStruct((B, S), jnp.int32),
        ],
    )(types, inds, style_vector, w1, b1, w2, b2)


def _bigtab_body(temb_ref, iemb_ref, pemb_ref, out_ref):
    g = pl.program_id(0)
    trow = jnp.zeros((1, D), jnp.float32)
    for t in range(NTYPE):
        trow = trow + temb_ref[pl.ds(t, 1), :] * jnp.where(g == t, 1.0, 0.0)
    big = (iemb_ref[...][:, None, :] + pemb_ref[...][None, :, :]
           + trow[None, :, :])
    out_ref[...] = big.reshape(NIDX * S, D)


def _tc_bigtab(temb, iemb, pemb):
    return pl.pallas_call(
        _bigtab_body,
        grid=(NTYPE,),
        in_specs=[
            pl.BlockSpec((NTYPE, D), lambda g: (0, 0)),
            pl.BlockSpec((NIDX, D), lambda g: (0, 0)),
            pl.BlockSpec((S, D), lambda g: (0, 0)),
        ],
        out_specs=pl.BlockSpec((NIDX * S, D), lambda g: (g, 0)),
        out_shape=jax.ShapeDtypeStruct((NTAB, D), jnp.float32),
    )(temb, iemb, pemb)


@functools.partial(
    pl.kernel,
    out_type=jax.ShapeDtypeStruct((B * S, D), jnp.float32),
    mesh=plsc.VectorSubcoreMesh(
        core_axis_name="c", subcore_axis_name="s",
        num_cores=NC, num_subcores=NS),
    compiler_params=pltpu.CompilerParams(needs_layout_passes=False),
    scratch_types=[
        pltpu.VMEM((S, D), jnp.float32),
        pltpu.VMEM((S, D), jnp.float32),
        pltpu.VMEM((3 * MSTR,), jnp.int32),
        pltpu.VMEM((3 * D,), jnp.float32),
        pltpu.VMEM((MSTR + LANES,), jnp.int32),
        pltpu.SemaphoreType.DMA,
        pltpu.SemaphoreType.DMA,
        pltpu.SemaphoreType.DMA,
        pltpu.SemaphoreType.DMA,
        pltpu.SemaphoreType.DMA,
    ],
)
def _sc_embed(gidx_hbm, styled_hbm, tab_hbm, out_hbm,
              blk0, blk1, gidxb, styb, pglist,
              gsem0, gsem1, osem0, osem1, fsem):
    wid = lax.axis_index("s") * NC + lax.axis_index("c")
    b0 = wid * NB
    iota = lax.iota(jnp.int32, LANES)

    def meta_src(r):
        return (gidx_hbm.at[pl.ds((b0 + r) * S, S)],
                styled_hbm.at[pl.ds((b0 + r) * D, D)])

    def meta_dst(r):
        m = (r % 3)
        return (gidxb.at[pl.ds(m * MSTR, S)], styb.at[pl.ds(m * D, D)])

    def gather_pair(r, blk):
        m = (r % 3) * MSTR
        return ((tab_hbm.at[gidxb.at[pl.ds(m, G0)]], blk.at[pl.ds(0, G0)]),
                (tab_hbm.at[gidxb.at[pl.ds(m + G0, G1)]],
                 blk.at[pl.ds(G0, G1)]))

    for r in (0, 1):
        for sx, dx in zip(meta_src(r), meta_dst(r)):
            pltpu.sync_copy(sx, dx)
    for sx, dx in gather_pair(0, blk0):
        pltpu.async_copy(sx, dx, gsem0)

    def do_row(r, p, blk_p, blk_o, gsem_p, gsem_o, osem_p, osem_o):
        mb = (r % 3)

        @pl.when((r >= 1) & (r + 1 < NB))
        def _():
            for sx, dx in zip(meta_src(r + 1), meta_dst(r + 1)):
                pltpu.make_async_copy(sx, dx, fsem).wait()

        @pl.when(r + 2 < NB)
        def _():
            for sx, dx in zip(meta_src(r + 2), meta_dst(r + 2)):
                pltpu.async_copy(sx, dx, fsem)

        @pl.when(r >= 1)
        def _():
            pltpu.make_async_copy(
                blk_o, out_hbm.at[pl.ds((b0 + r - 1) * S, S)], osem_o).wait()

        @pl.when(r + 1 < NB)
        def _():
            for sx, dx in gather_pair(r + 1, blk_o):
                pltpu.async_copy(sx, dx, gsem_o)

        for sx, dx in gather_pair(r, blk_p):
            pltpu.make_async_copy(sx, dx, gsem_p).wait()

        sty = tuple(styb[pl.ds(mb * D + k * LANES, LANES)]
                    for k in range(D // LANES))
        cnt = 0
        for w in range(S // LANES + 1):
            off = w * LANES
            gv = gidxb[pl.ds(mb * MSTR + off, LANES)]
            pm = (gv >= PAGE_LO) & (gv < PAGE_HI)
            if w == S // LANES:
                pm = pm & (iota < S - off)
            plsc.store_compressed(pglist.at[pl.ds(cnt, LANES)],
                                  iota + off, mask=pm)
            cnt = cnt + plsc.all_reduce_population_count(pm)[0]

        def fix_body(wi, carry):
            pg = pglist[pl.ds(wi * LANES, LANES)]
            for l in range(LANES):
                s_l = pg[l]

                @pl.when(wi * LANES + l < cnt)
                def _(_s=s_l):
                    for k in range(D // LANES):
                        blk_p[_s, pl.ds(k * LANES, LANES)] += sty[k]
            return carry

        lax.fori_loop(0, (cnt + LANES - 1) // LANES, fix_body, 0)

        pltpu.async_copy(blk_p, out_hbm.at[pl.ds((b0 + r) * S, S)], osem_p)

    def pair_body(h, carry):
        do_row(2 * h, 0, blk0, blk1, gsem0, gsem1, osem0, osem1)
        do_row(2 * h + 1, 1, blk1, blk0, gsem1, gsem0, osem1, osem0)
        return carry

    lax.fori_loop(0, NB // 2, pair_body, 0)

    pltpu.make_async_copy(blk1, out_hbm.at[pl.ds((b0 + NB - 1) * S, S)],
                          osem1).wait()


def kernel(element_types, element_indices, style_vector, type_emb, idx_emb,
           W1, b1, W2, b2, pos_emb):
    types = element_types.astype(jnp.int32)
    inds = element_indices.astype(jnp.int32)
    styled, mask, gidx = _tc_pre(types, inds, style_vector, W1, b1, W2, b2)
    bigtab = _tc_bigtab(type_emb, idx_emb, pos_emb)
    final = _sc_embed(gidx.reshape(-1), styled.reshape(-1), bigtab)
    return final.reshape(B, S, D), mask
